# baseline (speedup 1.0000x reference)
import functools

import jax
import jax.numpy as jnp
from jax import lax
from jax.experimental import pallas as pl
from jax.experimental.pallas import tpu as pltpu
from jax.experimental.pallas import tpu_sc as plsc

_HOPS = 10
_NC = 2
_NS = 16
_LANES = 16
_CHUNK = 128
_GRP = 8

_DNUMS = jax.lax.GatherDimensionNumbers(
    offset_dims=(), collapsed_slice_dims=(0,), start_index_map=(0,))


def _splat(v, lane):
    idx = jnp.full((16, 1), lane, jnp.int32)
    return jax.lax.gather(v, idx, _DNUMS, (1,),
                          mode=jax.lax.GatherScatterMode.PROMISE_IN_BOUNDS)


@functools.lru_cache(maxsize=None)
def _build(n, d, cap):
    dh = d // _NC
    nvec = dh // _LANES
    n_pad = -(-n // (_NS * _CHUNK)) * (_NS * _CHUNK)
    rows_tile = n_pad // _NS
    nk = rows_tile // _CHUNK

    mesh = plsc.VectorSubcoreMesh(core_axis_name="c", subcore_axis_name="s")

    def body(xr, col2, row2, ev2, gam, meta, h, outr,
             col8_v, row8_v, ev8_v, rows4, acc_l, out_acc, gam_v, meta_v,
             gsem):
        c = lax.axis_index("c")
        s = lax.axis_index("s")
        cn = c * n_pad
        ra = s * rows_tile

        pltpu.sync_copy(gam, gam_v)
        pltpu.sync_copy(meta, meta_v)
        cnt_s = jnp.max(meta_v[s, :])
        off_s = jnp.max(meta_v[s + _NS, :])
        ng = cnt_s // _GRP
        zv = jnp.zeros((_LANES,), jnp.float32)
        lanes_iota = jnp.arange(_LANES, dtype=jnp.int32)
        colidx = [lanes_iota + jf * _LANES for jf in range(nvec)]
        drain_src = xr.at[pl.ds(0, _CHUNK)]

        g0 = gam_v[0, :]
        for k in range(nk):
            pltpu.sync_copy(xr.at[pl.ds(cn + ra + k * _CHUNK, _CHUNK)],
                            rows4.at[0])
            def initk(r, carry, _k=k):
                for jf in range(nvec):
                    sl = pl.ds(jf * _LANES, _LANES)
                    out_acc[_k * _CHUNK + r, sl] = g0 * rows4[0, r, sl]
                return carry
            lax.fori_loop(0, _CHUNK, initk, 0)

        def zero_acc():
            @plsc.parallel_loop(0, rows_tile, unroll=4)
            def za(r):
                for jf in range(nvec):
                    acc_l[r, pl.ds(jf * _LANES, _LANES)] = zv

        def edge_phase(src, add_cn):
            def process(j, buf):
                def grp16(g, carry):
                    evv = ev8_v[j, pl.ds(g * _LANES, _LANES)]
                    rvv = row8_v[j, pl.ds(g * _LANES, _LANES)]
                    for lane in range(_LANES):
                        evb = _splat(evv, lane)
                        riv = _splat(rvv, lane)
                        e = g * _LANES + lane
                        for jf in range(nvec):
                            sl = pl.ds(jf * _LANES, _LANES)
                            plsc.addupdate_scatter(
                                acc_l, [riv, colidx[jf]], evb * buf[e, sl])
                    return carry
                lax.fori_loop(0, _CHUNK // _LANES, grp16, 0)

            def group(gi, carry):
                gb = off_s + gi * _GRP
                pltpu.sync_copy(col2.at[pl.ds(gb, _GRP)], col8_v)
                pltpu.sync_copy(row2.at[pl.ds(gb, _GRP)], row8_v)
                pltpu.sync_copy(ev2.at[pl.ds(gb, _GRP)], ev8_v)

                @plsc.parallel_loop(0, _GRP, unroll=2)
                def adjj(j):
                    for m in range(_CHUNK // _LANES):
                        sl = pl.ds(m * _LANES, _LANES)
                        col8_v[j, sl] = col8_v[j, sl] + cn

                pltpu.async_copy(src.at[col8_v.at[0]], rows4.at[0], gsem.at[0])
                pltpu.async_copy(src.at[col8_v.at[1]], rows4.at[1], gsem.at[1])

                def chunk(j, c2):
                    sl4 = jnp.bitwise_and(j, 3)
                    nj = j + 2
                    s2 = jnp.bitwise_and(nj, 3)
                    @pl.when(nj < _GRP)
                    def _():
                        pltpu.async_copy(src.at[col8_v.at[nj]], rows4.at[s2],
                                         gsem.at[s2])
                    buf = rows4.at[sl4]
                    pltpu.make_async_copy(drain_src, buf, gsem.at[sl4]).wait()
                    process(j, buf)
                    return c2
                lax.fori_loop(0, _GRP, chunk, 0)
                return carry
            lax.fori_loop(0, ng, group, 0)

        def writeback(l, write_h):
            g = gam_v[l, :]
            @pl.when(write_h)
            def _():
                pltpu.sync_copy(acc_l, h.at[pl.ds(cn + ra, rows_tile)])
            @plsc.parallel_loop(0, rows_tile, unroll=2)
            def wb(r):
                for jf in range(nvec):
                    sl = pl.ds(jf * _LANES, _LANES)
                    out_acc[r, sl] = out_acc[r, sl] + g * acc_l[r, sl]

        zero_acc()
        edge_phase(xr, True)
        plsc.subcore_barrier()
        writeback(1, jnp.bool_(True))
        zero_acc()
        plsc.subcore_barrier()

        def hop(l, carry):
            edge_phase(h, False)
            plsc.subcore_barrier()
            writeback(l, l < _HOPS)
            zero_acc()
            plsc.subcore_barrier()
            return carry
        lax.fori_loop(2, _HOPS + 1, hop, 0)

        pltpu.sync_copy(out_acc, outr.at[pl.ds(cn + ra, rows_tile)])

    f32 = jnp.float32
    i32 = jnp.int32
    return pl.kernel(
        body,
        out_type=(
            jax.ShapeDtypeStruct((_NC * n_pad, dh), f32),
            jax.ShapeDtypeStruct((_NC * n_pad, dh), f32),
        ),
        mesh=mesh,
        compiler_params=pltpu.CompilerParams(use_tc_tiling_on_sc=False,
                                             needs_layout_passes=False),
        scratch_types=[
            pltpu.VMEM((_GRP, _CHUNK), i32),
            pltpu.VMEM((_GRP, _CHUNK), i32),
            pltpu.VMEM((_GRP, _CHUNK), f32),
            pltpu.VMEM((4, _CHUNK, dh), f32),
            pltpu.VMEM((rows_tile, dh), f32),
            pltpu.VMEM((rows_tile, dh), f32),
            pltpu.VMEM((_LANES, _LANES), f32),
            pltpu.VMEM((2 * _NS, _LANES), i32),
            pltpu.SemaphoreType.DMA((4,)),
        ],
    )


def kernel(X, edge_index, edge_values, gpr_weights):
    n, d = X.shape
    e = edge_values.shape[0]
    dh = d // _NC
    n_pad = -(-n // (_NS * _CHUNK)) * (_NS * _CHUNK)
    rows_tile = n_pad // _NS
    cap = e // _CHUNK + _NS * _GRP
    i32 = jnp.int32

    row = edge_index[0]
    order = jnp.argsort(row)
    rs = jnp.take(row, order)
    cs = jnp.take(edge_index[1], order)
    evs = jnp.take(edge_values, order)
    tt = rs // rows_tile
    bnd = jnp.searchsorted(
        rs, (jnp.arange(_NS + 1, dtype=i32) * rows_tile).astype(rs.dtype))
    seg = bnd[1:] - bnd[:-1]
    cnt8 = (-(-(-(-seg // _CHUNK)) // _GRP)) * _GRP
    coff = jnp.concatenate(
        [jnp.zeros((1,), cnt8.dtype), jnp.cumsum(cnt8)])[:-1]
    dest = coff[tt] * _CHUNK + (jnp.arange(e) - bnd[tt])
    colp = jnp.zeros((cap * _CHUNK,), i32).at[dest].set(
        cs, unique_indices=True, indices_are_sorted=True)
    rowp = jnp.zeros((cap * _CHUNK,), i32).at[dest].set(
        (rs - tt * rows_tile).astype(i32),
        unique_indices=True, indices_are_sorted=True)
    evp = jnp.zeros((cap * _CHUNK,), jnp.float32).at[dest].set(
        evs, unique_indices=True, indices_are_sorted=True)
    meta = jnp.tile(
        jnp.concatenate([cnt8, coff]).astype(i32)[:, None], (1, _LANES))

    xr = jnp.pad(X.reshape(n, _NC, dh).transpose(1, 0, 2),
                 ((0, 0), (0, n_pad - n), (0, 0))).reshape(_NC * n_pad, dh)
    gam = jnp.zeros((_LANES, _LANES), jnp.float32).at[
        :gpr_weights.shape[0]].set(gpr_weights[:, None])

    _, outr = _build(n, d, cap)(
        xr, colp.reshape(cap, _CHUNK), rowp.reshape(cap, _CHUNK),
        evp.reshape(cap, _CHUNK), gam, meta)
    return outr.reshape(_NC, n_pad, dh)[:, :n].transpose(1, 0, 2).reshape(n, d)


# speedup vs baseline: 1.3172x; 1.3172x over previous
import functools

import jax
import jax.numpy as jnp
from jax import lax
from jax.experimental import pallas as pl
from jax.experimental.pallas import tpu as pltpu
from jax.experimental.pallas import tpu_sc as plsc

_HOPS = 10
_NC = 2
_NS = 16
_LANES = 16
_CHUNK = 128
_GRP = 8

_DNUMS = jax.lax.GatherDimensionNumbers(
    offset_dims=(), collapsed_slice_dims=(0,), start_index_map=(0,))


def _splat(v, lane):
    idx = jnp.full((16, 1), lane, jnp.int32)
    return jax.lax.gather(v, idx, _DNUMS, (1,),
                          mode=jax.lax.GatherScatterMode.PROMISE_IN_BOUNDS)


@functools.lru_cache(maxsize=None)
def _build(n, d, cap):
    dh = d // _NC
    nvec = dh // _LANES
    n_pad = -(-n // (_NS * _CHUNK)) * (_NS * _CHUNK)
    rows_tile = n_pad // _NS
    nk = rows_tile // _CHUNK

    mesh = plsc.VectorSubcoreMesh(core_axis_name="c", subcore_axis_name="s")

    def body(xr, col2, row2, ev2, gam, meta, h, outr,
             col8_v, row8_v, ev8_v, rows4, acc_l, out_acc, gam_v, meta_v,
             gsem):
        c = lax.axis_index("c")
        s = lax.axis_index("s")
        cn = c * n_pad
        ra = s * rows_tile

        pltpu.sync_copy(gam, gam_v)
        pltpu.sync_copy(meta, meta_v)
        cnt_s = jnp.max(meta_v[s, :])
        off_s = jnp.max(meta_v[s + _NS, :])
        ng = cnt_s // _GRP
        zv = jnp.zeros((_LANES,), jnp.float32)
        lanes_iota = jnp.arange(_LANES, dtype=jnp.int32)
        colidx = [lanes_iota + jf * _LANES for jf in range(nvec)]
        drain_src = xr.at[pl.ds(0, _CHUNK)]

        g0 = gam_v[0, :]
        for k in range(nk):
            pltpu.sync_copy(xr.at[pl.ds(cn + ra + k * _CHUNK, _CHUNK)],
                            rows4.at[0])
            def initk(r, carry, _k=k):
                for jf in range(nvec):
                    sl = pl.ds(jf * _LANES, _LANES)
                    out_acc[_k * _CHUNK + r, sl] = g0 * rows4[0, r, sl]
                return carry
            lax.fori_loop(0, _CHUNK, initk, 0)

        def zero_acc():
            @plsc.parallel_loop(0, rows_tile, unroll=4)
            def za(r):
                for jf in range(nvec):
                    acc_l[r, pl.ds(jf * _LANES, _LANES)] = zv

        def edge_phase(src, add_cn):
            def process(j, buf):
                @plsc.parallel_loop(0, _CHUNK // _LANES, unroll=2)
                def grp16(g):
                    evv = ev8_v[j, pl.ds(g * _LANES, _LANES)]
                    rvv = row8_v[j, pl.ds(g * _LANES, _LANES)]
                    for lane in range(_LANES):
                        evb = _splat(evv, lane)
                        riv = _splat(rvv, lane)
                        e = g * _LANES + lane
                        for jf in range(nvec):
                            sl = pl.ds(jf * _LANES, _LANES)
                            plsc.addupdate_scatter(
                                acc_l, [riv, colidx[jf]], evb * buf[e, sl])

            def group(gi, carry):
                gb = off_s + gi * _GRP
                pltpu.sync_copy(col2.at[pl.ds(gb, _GRP)], col8_v)
                pltpu.sync_copy(row2.at[pl.ds(gb, _GRP)], row8_v)
                pltpu.sync_copy(ev2.at[pl.ds(gb, _GRP)], ev8_v)

                @plsc.parallel_loop(0, _GRP, unroll=2)
                def adjj(j):
                    for m in range(_CHUNK // _LANES):
                        sl = pl.ds(m * _LANES, _LANES)
                        col8_v[j, sl] = col8_v[j, sl] + cn

                pltpu.async_copy(src.at[col8_v.at[0]], rows4.at[0], gsem.at[0])
                pltpu.async_copy(src.at[col8_v.at[1]], rows4.at[1], gsem.at[1])

                def chunk(j, c2):
                    sl4 = jnp.bitwise_and(j, 3)
                    nj = j + 2
                    s2 = jnp.bitwise_and(nj, 3)
                    @pl.when(nj < _GRP)
                    def _():
                        pltpu.async_copy(src.at[col8_v.at[nj]], rows4.at[s2],
                                         gsem.at[s2])
                    buf = rows4.at[sl4]
                    pltpu.make_async_copy(drain_src, buf, gsem.at[sl4]).wait()
                    process(j, buf)
                    return c2
                lax.fori_loop(0, _GRP, chunk, 0)
                return carry
            lax.fori_loop(0, ng, group, 0)

        def writeback(l, write_h):
            g = gam_v[l, :]
            @pl.when(write_h)
            def _():
                pltpu.sync_copy(acc_l, h.at[pl.ds(cn + ra, rows_tile)])
            @plsc.parallel_loop(0, rows_tile, unroll=2)
            def wb(r):
                for jf in range(nvec):
                    sl = pl.ds(jf * _LANES, _LANES)
                    out_acc[r, sl] = out_acc[r, sl] + g * acc_l[r, sl]

        zero_acc()
        edge_phase(xr, True)
        plsc.subcore_barrier()
        writeback(1, jnp.bool_(True))
        zero_acc()
        plsc.subcore_barrier()

        def hop(l, carry):
            edge_phase(h, False)
            plsc.subcore_barrier()
            writeback(l, l < _HOPS)
            zero_acc()
            plsc.subcore_barrier()
            return carry
        lax.fori_loop(2, _HOPS + 1, hop, 0)

        pltpu.sync_copy(out_acc, outr.at[pl.ds(cn + ra, rows_tile)])

    f32 = jnp.float32
    i32 = jnp.int32
    return pl.kernel(
        body,
        out_type=(
            jax.ShapeDtypeStruct((_NC * n_pad, dh), f32),
            jax.ShapeDtypeStruct((_NC * n_pad, dh), f32),
        ),
        mesh=mesh,
        compiler_params=pltpu.CompilerParams(use_tc_tiling_on_sc=False,
                                             needs_layout_passes=False),
        scratch_types=[
            pltpu.VMEM((_GRP, _CHUNK), i32),
            pltpu.VMEM((_GRP, _CHUNK), i32),
            pltpu.VMEM((_GRP, _CHUNK), f32),
            pltpu.VMEM((4, _CHUNK, dh), f32),
            pltpu.VMEM((rows_tile, dh), f32),
            pltpu.VMEM((rows_tile, dh), f32),
            pltpu.VMEM((_LANES, _LANES), f32),
            pltpu.VMEM((2 * _NS, _LANES), i32),
            pltpu.SemaphoreType.DMA((4,)),
        ],
    )


def kernel(X, edge_index, edge_values, gpr_weights):
    n, d = X.shape
    e = edge_values.shape[0]
    dh = d // _NC
    n_pad = -(-n // (_NS * _CHUNK)) * (_NS * _CHUNK)
    rows_tile = n_pad // _NS
    cap = e // _CHUNK + _NS * _GRP
    i32 = jnp.int32

    row = edge_index[0]
    order = jnp.argsort(row)
    rs = jnp.take(row, order)
    cs = jnp.take(edge_index[1], order)
    evs = jnp.take(edge_values, order)
    tt = rs // rows_tile
    bnd = jnp.searchsorted(
        rs, (jnp.arange(_NS + 1, dtype=i32) * rows_tile).astype(rs.dtype))
    seg = bnd[1:] - bnd[:-1]
    cnt8 = (-(-(-(-seg // _CHUNK)) // _GRP)) * _GRP
    coff = jnp.concatenate(
        [jnp.zeros((1,), cnt8.dtype), jnp.cumsum(cnt8)])[:-1]
    dest = coff[tt] * _CHUNK + (jnp.arange(e) - bnd[tt])
    colp = jnp.zeros((cap * _CHUNK,), i32).at[dest].set(
        cs, unique_indices=True, indices_are_sorted=True)
    rowp = jnp.zeros((cap * _CHUNK,), i32).at[dest].set(
        (rs - tt * rows_tile).astype(i32),
        unique_indices=True, indices_are_sorted=True)
    evp = jnp.zeros((cap * _CHUNK,), jnp.float32).at[dest].set(
        evs, unique_indices=True, indices_are_sorted=True)
    meta = jnp.tile(
        jnp.concatenate([cnt8, coff]).astype(i32)[:, None], (1, _LANES))

    xr = jnp.pad(X.reshape(n, _NC, dh).transpose(1, 0, 2),
                 ((0, 0), (0, n_pad - n), (0, 0))).reshape(_NC * n_pad, dh)
    gam = jnp.zeros((_LANES, _LANES), jnp.float32).at[
        :gpr_weights.shape[0]].set(gpr_weights[:, None])

    _, outr = _build(n, d, cap)(
        xr, colp.reshape(cap, _CHUNK), rowp.reshape(cap, _CHUNK),
        evp.reshape(cap, _CHUNK), gam, meta)
    return outr.reshape(_NC, n_pad, dh)[:, :n].transpose(1, 0, 2).reshape(n, d)
